# baseline (speedup 1.0000x reference)
import functools

import jax
import jax.numpy as jnp
from jax import lax
from jax.experimental import pallas as pl
from jax.experimental.pallas import tpu as pltpu
from jax.experimental.pallas import tpu_sc as plsc

D_MODEL = 64
SCALE = 8.0

_IPG = 128
_GPC = 8
_CH = _IPG * _GPC


@functools.lru_cache(maxsize=None)
def _make_sc_gather(B: int, V: int):
    info = plsc.get_sparse_core_info()
    NC, NS = info.num_cores, info.num_subcores
    NW = NC * NS
    assert B % (NW * _CH) == 0, B
    chunks_per_w = B // (NW * _CH)
    rows_per_w = B // NW

    mesh = plsc.VectorSubcoreMesh(core_axis_name="c", subcore_axis_name="s")

    @functools.partial(
        pl.kernel,
        mesh=mesh,
        out_type=jax.ShapeDtypeStruct((B, D_MODEL), jnp.float32),
        scratch_types=[
            pltpu.VMEM((_GPC, _IPG), jnp.int32),
            pltpu.VMEM((_CH, D_MODEL), jnp.float32),
            pltpu.SemaphoreType.DMA,
        ],
        compiler_params=pltpu.CompilerParams(use_tc_tiling_on_sc=False),
    )
    def k(emb_hbm, idx_hbm, out_hbm, idx_v, rows_v, sem):
        wid = lax.axis_index("s") * NC + lax.axis_index("c")
        w_base = wid * rows_per_w

        def chunk_body(g, carry):
            base = w_base + g * _CH
            idx_row = pl.multiple_of(base // _IPG, 8)
            pltpu.sync_copy(idx_hbm.at[pl.ds(idx_row, _GPC)], idx_v)
            copies = []
            for j in range(_GPC):
                copies.append(
                    pltpu.async_copy(
                        emb_hbm.at[idx_v.at[j]],
                        rows_v.at[pl.ds(j * _IPG, _IPG)],
                        sem,
                    )
                )
            for c in copies:
                c.wait()

            def scale_body(i, carry2):
                for j in range(D_MODEL // 16):
                    sl = pl.ds(j * 16, 16)
                    rows_v[i, sl] = rows_v[i, sl] * SCALE
                return carry2

            lax.fori_loop(0, _CH, scale_body, 0, unroll=4)

            pltpu.sync_copy(rows_v, out_hbm.at[pl.ds(base, _CH)])
            return carry

        lax.fori_loop(0, chunks_per_w, chunk_body, 0)

    return k


def kernel(x, emb):
    B = x.size
    xf = x.reshape(B // _IPG, _IPG).astype(jnp.int32)
    out = _make_sc_gather(B, emb.shape[0])(emb, xf)
    return out.reshape(x.shape + (D_MODEL,))


# speedup vs baseline: 1.0473x; 1.0473x over previous
import functools

import jax
import jax.numpy as jnp
from jax import lax
from jax.experimental import pallas as pl
from jax.experimental.pallas import tpu as pltpu
from jax.experimental.pallas import tpu_sc as plsc

D_MODEL = 64
SCALE = 8.0

_IPG = 128
_GPC = 4
_CH = _IPG * _GPC


@functools.lru_cache(maxsize=None)
def _make_sc_gather(B: int, V: int):
    info = plsc.get_sparse_core_info()
    NC, NS = info.num_cores, info.num_subcores
    NW = NC * NS
    assert B % (NW * _CH) == 0, B
    chunks_per_w = B // (NW * _CH)
    rows_per_w = B // NW

    mesh = plsc.VectorSubcoreMesh(core_axis_name="c", subcore_axis_name="s")

    @functools.partial(
        pl.kernel,
        mesh=mesh,
        out_type=jax.ShapeDtypeStruct((B, D_MODEL), jnp.float32),
        scratch_types=[
            pltpu.VMEM((2, _CH), jnp.int32),
            pltpu.VMEM((2, _CH, D_MODEL), jnp.float32),
            pltpu.SemaphoreType.DMA,
            pltpu.SemaphoreType.DMA,
        ],
        compiler_params=pltpu.CompilerParams(use_tc_tiling_on_sc=False),
    )
    def k(emb_hbm, idx_hbm, out_hbm, idx_v, rows_v, sem0, sem1):
        sems = (sem0, sem1)
        wid = lax.axis_index("s") * NC + lax.axis_index("c")
        w_chunk0 = wid * chunks_per_w

        def fire(c, p):
            base = pl.multiple_of((w_chunk0 + c) * _CH, 8)
            pltpu.sync_copy(idx_hbm.at[pl.ds(base, _CH)], idx_v.at[p])
            for j in range(_GPC):
                pltpu.async_copy(
                    emb_hbm.at[idx_v.at[p, pl.ds(j * _IPG, _IPG)]],
                    rows_v.at[p, pl.ds(j * _IPG, _IPG)],
                    sems[p],
                )

        def drain(p):
            for j in range(_GPC):
                pltpu.make_async_copy(
                    emb_hbm.at[idx_v.at[p, pl.ds(j * _IPG, _IPG)]],
                    rows_v.at[p, pl.ds(j * _IPG, _IPG)],
                    sems[p],
                ).wait()

        def finish(c, p):
            drain(p)

            @plsc.parallel_loop(0, _CH, unroll=8)
            def _scale(i):
                for j in range(D_MODEL // 16):
                    sl = pl.ds(j * 16, 16)
                    rows_v[p, i, sl] = rows_v[p, i, sl] * SCALE

            base = (w_chunk0 + c) * _CH
            pltpu.sync_copy(rows_v.at[p], out_hbm.at[pl.ds(base, _CH)])

        fire(0, 0)

        def pair_body(i, carry):
            for b in range(2):
                c = 2 * i + b
                fire(c + 1, 1 - b)
                finish(c, b)
            return carry

        n = chunks_per_w
        lax.fori_loop(0, (n - 1) // 2, pair_body, 0)
        if n % 2 == 1:
            finish(n - 1, (n - 1) % 2)
        else:
            fire(n - 1, (n - 1) % 2)
            finish(n - 2, (n - 2) % 2)
            finish(n - 1, (n - 1) % 2)

    return k


def kernel(x, emb):
    B = x.size
    xf = x.reshape(B).astype(jnp.int32)
    out = _make_sc_gather(B, emb.shape[0])(emb, xf)
    return out.reshape(x.shape + (D_MODEL,))


# speedup vs baseline: 1.0589x; 1.0111x over previous
import functools

import jax
import jax.numpy as jnp
from jax import lax
from jax.experimental import pallas as pl
from jax.experimental.pallas import tpu as pltpu
from jax.experimental.pallas import tpu_sc as plsc

D_MODEL = 64
SCALE = 8.0
_S = 4


@functools.lru_cache(maxsize=None)
def _make_sc_gather(R: int, C: int, V: int):
    info = plsc.get_sparse_core_info()
    NC, NS = info.num_cores, info.num_subcores
    NW = NC * NS
    assert R % (NW * _S) == 0, R
    chunks_per_w = R // (NW * _S)
    xrows_per_w = R // NW
    splits = []
    o = 0
    while o < C:
        splits.append((o, min(128, C - o)))
        o += 128

    mesh = plsc.VectorSubcoreMesh(core_axis_name="c", subcore_axis_name="s")

    @functools.partial(
        pl.kernel,
        mesh=mesh,
        out_type=jax.ShapeDtypeStruct((R, C, D_MODEL), jnp.float32),
        scratch_types=[
            pltpu.VMEM((2, _S, C), jnp.int32),
            pltpu.VMEM((2, _S, C, D_MODEL), jnp.float32),
            pltpu.SemaphoreType.DMA,
            pltpu.SemaphoreType.DMA,
        ],
        compiler_params=pltpu.CompilerParams(use_tc_tiling_on_sc=False),
    )
    def k(emb_hbm, idx_hbm, out_hbm, idx_v, rows_v, sem0, sem1):
        sems = (sem0, sem1)
        wid = lax.axis_index("s") * NC + lax.axis_index("c")
        w_xrow0 = wid * xrows_per_w

        def fire(c, p):
            xrow = w_xrow0 + c * _S
            pltpu.sync_copy(idx_hbm.at[pl.ds(xrow, _S)], idx_v.at[p])
            for r in range(_S):
                for o, n in splits:
                    pltpu.async_copy(
                        emb_hbm.at[idx_v.at[p, r, pl.ds(o, n)]],
                        rows_v.at[p, r, pl.ds(o, n)],
                        sems[p],
                    )

        def drain(p):
            for r in range(_S):
                for o, n in splits:
                    pltpu.make_async_copy(
                        emb_hbm.at[idx_v.at[p, r, pl.ds(o, n)]],
                        rows_v.at[p, r, pl.ds(o, n)],
                        sems[p],
                    ).wait()

        def finish(c, p):
            drain(p)

            @plsc.parallel_loop(0, C, unroll=4)
            def _scale(i):
                for r in range(_S):
                    for j in range(D_MODEL // 16):
                        sl = pl.ds(j * 16, 16)
                        rows_v[p, r, i, sl] = rows_v[p, r, i, sl] * SCALE

            xrow = w_xrow0 + c * _S
            pltpu.sync_copy(rows_v.at[p], out_hbm.at[pl.ds(xrow, _S)])

        fire(0, 0)

        def pair_body(i, carry):
            for b in range(2):
                c = 2 * i + b
                fire(c + 1, 1 - b)
                finish(c, b)
            return carry

        n = chunks_per_w
        lax.fori_loop(0, (n - 1) // 2, pair_body, 0)
        if n % 2 == 1:
            finish(n - 1, (n - 1) % 2)
        else:
            fire(n - 1, (n - 1) % 2)
            finish(n - 2, (n - 2) % 2)
            finish(n - 1, (n - 1) % 2)

    return k


def kernel(x, emb):
    xi = x.astype(jnp.int32)
    return _make_sc_gather(x.shape[0], x.shape[1], emb.shape[0])(emb, xi)
